# baseline (speedup 1.0000x reference)
import jax
import jax.numpy as jnp
from jax import lax
from jax.experimental import pallas as pl
from jax.experimental.pallas import tpu as pltpu
from jax.experimental.pallas import tpu_sc as plsc

ROWS = 8192
CH = 2048
L = 16
NC = 2
NS = 16
NW = NC * NS
ROWS_PER_W = ROWS // NW
RI = 16
RO = 8
N_ICHUNKS = ROWS_PER_W // RI
N_JC = CH // L


def _body(x_hbm, perm_hbm, out_hbm, perm_v, in0_v, in1_v, out0_v, out1_v,
          sem_in0, sem_in1, sem_out0, sem_out1):
    wid = lax.axis_index("c") * NS + lax.axis_index("s")
    base = wid * ROWS_PER_W
    ins = (in0_v, in1_v)
    outs = (out0_v, out1_v)
    sem_ins = (sem_in0, sem_in1)
    sem_outs = (sem_out0, sem_out1)

    def start_in(ii, b):
        pltpu.async_copy(x_hbm.at[pl.ds(base + ii * RI, RI)], ins[b],
                         sem_ins[b])

    def wait_in(b):
        pltpu.make_async_copy(x_hbm.at[pl.ds(base, RI)], ins[b],
                              sem_ins[b]).wait()

    def start_out(oi, b):
        pltpu.async_copy(outs[b], out_hbm.at[pl.ds(base + oi * RO, RO)],
                         sem_outs[b])

    def wait_out(b):
        pltpu.make_async_copy(outs[b], out_hbm.at[pl.ds(base, RO)],
                              sem_outs[b]).wait()

    start_in(0, 0)
    pltpu.sync_copy(perm_hbm, perm_v)

    def ichunk(p, carry):
        for b in range(2):
            ii = 2 * p + b
            wait_in(b)

            @pl.when(ii + 1 < N_ICHUNKS)
            def _():
                start_in(ii + 1, 1 - b)

            in_v = ins[b]
            for h in range(2):
                oi = 2 * ii + h

                @pl.when(oi >= 2)
                def _():
                    wait_out(h)

                out_v = outs[h]

                @plsc.parallel_loop(0, N_JC, unroll=2)
                def _col(j):
                    idx = perm_v[pl.ds(j * L, L)]
                    for r in range(RO):
                        rvec = jnp.full((L,), h * RO + r, jnp.int32)
                        out_v[r, pl.ds(j * L, L)] = plsc.load_gather(
                            in_v, [rvec, idx])

                start_out(oi, h)
        return carry

    lax.fori_loop(0, N_ICHUNKS // 2, ichunk, 0)
    wait_out(0)
    wait_out(1)


@jax.jit
def kernel(x, perm):
    f = pl.kernel(
        _body,
        out_type=jax.ShapeDtypeStruct((ROWS, CH), jnp.float32),
        mesh=plsc.VectorSubcoreMesh(core_axis_name="c", subcore_axis_name="s"),
        scratch_types=[
            pltpu.VMEM((CH,), jnp.int32),
            pltpu.VMEM((RI, CH), jnp.float32),
            pltpu.VMEM((RI, CH), jnp.float32),
            pltpu.VMEM((RO, CH), jnp.float32),
            pltpu.VMEM((RO, CH), jnp.float32),
            pltpu.SemaphoreType.DMA,
            pltpu.SemaphoreType.DMA,
            pltpu.SemaphoreType.DMA,
            pltpu.SemaphoreType.DMA,
        ],
        compiler_params=pltpu.CompilerParams(needs_layout_passes=False),
    )
    return f(x, perm)


# speedup vs baseline: 1.0063x; 1.0063x over previous
import jax
import jax.numpy as jnp
from jax import lax
from jax.experimental import pallas as pl
from jax.experimental.pallas import tpu as pltpu
from jax.experimental.pallas import tpu_sc as plsc

ROWS = 8192
CH = 2048
L = 16
NC = 2
NS = 16
NW = NC * NS
ROWS_PER_W = ROWS // NW
RI = 16
RO = 8
N_ICHUNKS = ROWS_PER_W // RI
N_JC = CH // L


def _body(x_hbm, perm_hbm, out_hbm, perm_v, in0_v, in1_v, out0_v, out1_v,
          sem_in0, sem_in1, sem_out0, sem_out1):
    wid = lax.axis_index("c") * NS + lax.axis_index("s")
    base = wid * ROWS_PER_W
    ins = (in0_v, in1_v)
    outs = (out0_v, out1_v)
    sem_ins = (sem_in0, sem_in1)
    sem_outs = (sem_out0, sem_out1)

    def start_in(ii, b):
        pltpu.async_copy(x_hbm.at[pl.ds(base + ii * RI, RI)], ins[b],
                         sem_ins[b])

    def wait_in(b):
        pltpu.make_async_copy(x_hbm.at[pl.ds(base, RI)], ins[b],
                              sem_ins[b]).wait()

    def start_out(oi, b):
        pltpu.async_copy(outs[b], out_hbm.at[pl.ds(base + oi * RO, RO)],
                         sem_outs[b])

    def wait_out(b):
        pltpu.make_async_copy(outs[b], out_hbm.at[pl.ds(base, RO)],
                              sem_outs[b]).wait()

    start_in(0, 0)
    pltpu.sync_copy(perm_hbm, perm_v)

    def ichunk(p, carry):
        for b in range(2):
            ii = 2 * p + b
            wait_in(b)

            @pl.when(ii + 1 < N_ICHUNKS)
            def _():
                start_in(ii + 1, 1 - b)

            in_v = ins[b]
            for h in range(2):
                oi = 2 * ii + h

                @pl.when(oi >= 2)
                def _():
                    wait_out(h)

                out_v = outs[h]

                @plsc.parallel_loop(0, N_JC, unroll=4)
                def _col(j):
                    idx = perm_v[pl.ds(j * L, L)]
                    for r in range(RO):
                        rvec = jnp.full((L,), h * RO + r, jnp.int32)
                        out_v[r, pl.ds(j * L, L)] = plsc.load_gather(
                            in_v, [rvec, idx])

                start_out(oi, h)
        return carry

    lax.fori_loop(0, N_ICHUNKS // 2, ichunk, 0)
    wait_out(0)
    wait_out(1)


@jax.jit
def kernel(x, perm):
    f = pl.kernel(
        _body,
        out_type=jax.ShapeDtypeStruct((ROWS, CH), jnp.float32),
        mesh=plsc.VectorSubcoreMesh(core_axis_name="c", subcore_axis_name="s"),
        scratch_types=[
            pltpu.VMEM((CH,), jnp.int32),
            pltpu.VMEM((RI, CH), jnp.float32),
            pltpu.VMEM((RI, CH), jnp.float32),
            pltpu.VMEM((RO, CH), jnp.float32),
            pltpu.VMEM((RO, CH), jnp.float32),
            pltpu.SemaphoreType.DMA,
            pltpu.SemaphoreType.DMA,
            pltpu.SemaphoreType.DMA,
            pltpu.SemaphoreType.DMA,
        ],
        compiler_params=pltpu.CompilerParams(needs_layout_passes=False),
    )
    return f(x, perm)
